# Initial kernel scaffold; baseline (speedup 1.0000x reference)
import jax
import jax.numpy as jnp
from jax.experimental import pallas as pl


def kernel(x, edge_index, W1, W2):
    raise NotImplementedError("write your pallas kernel here")


# speedup vs baseline: 12.8406x; 12.8406x over previous
import functools

import jax
import jax.numpy as jnp
from jax import lax
from jax.experimental import pallas as pl
from jax.experimental.pallas import tpu as pltpu
from jax.experimental.pallas import tpu_sc as plsc

N_NODES = 10000
HID = 16
N_CLS = 40

NCORE = 2
NSUB = 16
NW = NCORE * NSUB
CHUNK = 128
NPAD = 10112
RPT = NPAD // NSUB


def _sc_spmm(feat, src3, dst3, cpt):
    mesh = plsc.VectorSubcoreMesh(core_axis_name="c", subcore_axis_name="s")

    @functools.partial(
        pl.kernel,
        out_type=jax.ShapeDtypeStruct((NCORE, NPAD, HID), jnp.float32),
        mesh=mesh,
        scratch_types=[
            pltpu.VMEM((cpt, CHUNK), jnp.int32),
            pltpu.VMEM((cpt, CHUNK), jnp.int32),
            pltpu.VMEM((CHUNK, HID), jnp.float32),
            pltpu.VMEM((RPT, HID), jnp.float32),
            pltpu.VMEM_SHARED((NPAD, HID), jnp.float32),
        ],
        compiler_params=pltpu.CompilerParams(use_tc_tiling_on_sc=False),
    )
    def spmm(feat_hbm, src_hbm, dst_hbm, out_hbm, src_v, dst_v, rows_v, buf_v, acc_sh):
        c = lax.axis_index("c")
        s = lax.axis_index("s")
        w = c * NSUB + s

        @pl.loop(0, RPT)
        def _(i):
            buf_v[i, :] = jnp.zeros((HID,), jnp.float32)

        pltpu.sync_copy(buf_v, acc_sh.at[pl.ds(s * RPT, RPT)])
        pltpu.sync_copy(src_hbm.at[w], src_v)
        pltpu.sync_copy(dst_hbm.at[w], dst_v)
        plsc.subcore_barrier()

        @pl.loop(0, cpt)
        def _(j):
            pltpu.sync_copy(feat_hbm.at[src_v.at[j]], rows_v)
            pltpu.sync_copy(rows_v, acc_sh.at[dst_v.at[j]], add=True)

        plsc.subcore_barrier()
        pltpu.sync_copy(acc_sh.at[pl.ds(s * RPT, RPT)], buf_v)
        pltpu.sync_copy(buf_v, out_hbm.at[c, pl.ds(s * RPT, RPT)])

    return spmm(feat, src3, dst3)


def _tc_in_proj(x, W1):
    def body(x_ref, w_ref, o_ref):
        o_ref[...] = jnp.dot(x_ref[...], w_ref[...],
                             preferred_element_type=jnp.float32)

    return pl.pallas_call(
        body,
        out_shape=jax.ShapeDtypeStruct((N_NODES, HID), jnp.float32),
    )(x, W1)


def _tc_add_relu(p):
    def body(p_ref, o_ref):
        o_ref[...] = jnp.maximum(p_ref[0, :N_NODES, :] + p_ref[1, :N_NODES, :],
                                 0.0)

    return pl.pallas_call(
        body,
        out_shape=jax.ShapeDtypeStruct((N_NODES, HID), jnp.float32),
    )(p)


def _tc_out_proj(q, W2):
    def body(q_ref, w_ref, o_ref):
        z = q_ref[0, :N_NODES, :] + q_ref[1, :N_NODES, :]
        logits = jnp.dot(z, w_ref[...], preferred_element_type=jnp.float32)
        m = jnp.max(logits, axis=1, keepdims=True)
        e = jnp.exp(logits - m)
        lse = jnp.log(jnp.sum(e, axis=1, keepdims=True)) + m
        o_ref[...] = logits - lse

    return pl.pallas_call(
        body,
        out_shape=jax.ShapeDtypeStruct((N_NODES, N_CLS), jnp.float32),
    )(q, W2)


def kernel(x, edge_index, W1, W2):
    n_edges = edge_index.shape[1]
    per_worker = (n_edges + NW - 1) // NW
    cpt = (per_worker + CHUNK - 1) // CHUNK
    epad = NW * cpt * CHUNK

    src = edge_index[0].astype(jnp.int32)
    dst = edge_index[1].astype(jnp.int32)
    pad = epad - n_edges
    src3 = jnp.concatenate(
        [src, jnp.zeros((pad,), jnp.int32)]).reshape(NW, cpt, CHUNK)
    dst3 = jnp.concatenate(
        [dst, jnp.full((pad,), N_NODES, jnp.int32)]).reshape(NW, cpt, CHUNK)

    t1 = _tc_in_proj(x, W1)
    p1 = _sc_spmm(t1, src3, dst3, cpt)
    h = _tc_add_relu(p1)
    p2 = _sc_spmm(h, src3, dst3, cpt)
    return _tc_out_proj(p2, W2)


# speedup vs baseline: 14.9806x; 1.1667x over previous
import functools

import jax
import jax.numpy as jnp
from jax import lax
from jax.experimental import pallas as pl
from jax.experimental.pallas import tpu as pltpu
from jax.experimental.pallas import tpu_sc as plsc

N_NODES = 10000
HID = 16
N_CLS = 40

NCORE = 2
NSUB = 16
NW = NCORE * NSUB
CHUNK = 125
NPAD = 10112
RPT = NPAD // NSUB


def _sc_spmm(feat, src3, dst3, cpt):
    mesh = plsc.VectorSubcoreMesh(core_axis_name="c", subcore_axis_name="s")

    assert cpt % 2 == 0, "double-buffered loop assumes an even chunk count"

    @functools.partial(
        pl.kernel,
        out_type=jax.ShapeDtypeStruct((NCORE, NPAD, HID), jnp.float32),
        mesh=mesh,
        scratch_types=[
            pltpu.VMEM((cpt, CHUNK), jnp.int32),
            pltpu.VMEM((cpt, CHUNK), jnp.int32),
            pltpu.VMEM((CHUNK, HID), jnp.float32),
            pltpu.VMEM((CHUNK, HID), jnp.float32),
            pltpu.VMEM((RPT, HID), jnp.float32),
            pltpu.VMEM_SHARED((NPAD, HID), jnp.float32),
            pltpu.SemaphoreType.DMA,
            pltpu.SemaphoreType.DMA,
            pltpu.SemaphoreType.DMA,
            pltpu.SemaphoreType.DMA,
        ],
        compiler_params=pltpu.CompilerParams(use_tc_tiling_on_sc=False),
    )
    def spmm(feat_hbm, src_hbm, dst_hbm, out_hbm, src_v, dst_v,
             rows0_v, rows1_v, buf_v, acc_sh, g0, g1, s0, s1):
        c = lax.axis_index("c")
        s = lax.axis_index("s")
        w = c * NSUB + s

        @pl.loop(0, RPT)
        def _(i):
            buf_v[i, :] = jnp.zeros((HID,), jnp.float32)

        pltpu.sync_copy(buf_v, acc_sh.at[pl.ds(s * RPT, RPT)])
        pltpu.sync_copy(src_hbm.at[w], src_v)
        pltpu.sync_copy(dst_hbm.at[w], dst_v)
        plsc.subcore_barrier()

        rows = (rows0_v, rows1_v)
        gsem = (g0, g1)
        ssem = (s0, s1)

        pltpu.async_copy(feat_hbm.at[src_v.at[0]], rows[0], gsem[0])

        @pl.loop(0, cpt, step=2)
        def _(j):
            for b in (0, 1):
                jj = j + b
                o = 1 - b
                pltpu.make_async_copy(
                    feat_hbm.at[src_v.at[jj]], rows[b], gsem[b]).wait()

                @pl.when(jj + 1 < cpt)
                def _():
                    @pl.when(jj >= 1)
                    def _():
                        pltpu.make_async_copy(
                            rows[o], acc_sh.at[dst_v.at[jj - 1]],
                            ssem[o]).wait()
                    pltpu.async_copy(
                        feat_hbm.at[src_v.at[jj + 1]], rows[o], gsem[o])

                pltpu.async_copy(
                    rows[b], acc_sh.at[dst_v.at[jj]], ssem[b], add=True)

        pltpu.make_async_copy(
            rows[0], acc_sh.at[dst_v.at[cpt - 2]], ssem[0]).wait()
        pltpu.make_async_copy(
            rows[1], acc_sh.at[dst_v.at[cpt - 1]], ssem[1]).wait()
        plsc.subcore_barrier()
        pltpu.sync_copy(acc_sh.at[pl.ds(s * RPT, RPT)], buf_v)
        pltpu.sync_copy(buf_v, out_hbm.at[c, pl.ds(s * RPT, RPT)])

    return spmm(feat, src3, dst3)


def _tc_in_proj(x, W1):
    def body(x_ref, w_ref, o_ref):
        o_ref[...] = jnp.dot(x_ref[...], w_ref[...],
                             preferred_element_type=jnp.float32)

    return pl.pallas_call(
        body,
        out_shape=jax.ShapeDtypeStruct((N_NODES, HID), jnp.float32),
    )(x, W1)


def _tc_add_relu(p):
    def body(p_ref, o_ref):
        o_ref[...] = jnp.maximum(p_ref[0, :N_NODES, :] + p_ref[1, :N_NODES, :],
                                 0.0)

    return pl.pallas_call(
        body,
        out_shape=jax.ShapeDtypeStruct((N_NODES, HID), jnp.float32),
    )(p)


def _tc_out_proj(q, W2):
    def body(q_ref, w_ref, o_ref):
        z = q_ref[0, :N_NODES, :] + q_ref[1, :N_NODES, :]
        logits = jnp.dot(z, w_ref[...], preferred_element_type=jnp.float32)
        m = jnp.max(logits, axis=1, keepdims=True)
        e = jnp.exp(logits - m)
        lse = jnp.log(jnp.sum(e, axis=1, keepdims=True)) + m
        o_ref[...] = logits - lse

    return pl.pallas_call(
        body,
        out_shape=jax.ShapeDtypeStruct((N_NODES, N_CLS), jnp.float32),
    )(q, W2)


def kernel(x, edge_index, W1, W2):
    n_edges = edge_index.shape[1]
    per_worker = (n_edges + NW - 1) // NW
    cpt = (per_worker + CHUNK - 1) // CHUNK
    cpt += cpt % 2
    epad = NW * cpt * CHUNK

    src = edge_index[0].astype(jnp.int32)
    dst = edge_index[1].astype(jnp.int32)
    pad = epad - n_edges
    if pad:
        src = jnp.concatenate([src, jnp.zeros((pad,), jnp.int32)])
        dst = jnp.concatenate([dst, jnp.full((pad,), N_NODES, jnp.int32)])
    src3 = src.reshape(NW, cpt, CHUNK)
    dst3 = dst.reshape(NW, cpt, CHUNK)

    t1 = _tc_in_proj(x, W1)
    p1 = _sc_spmm(t1, src3, dst3, cpt)
    h = _tc_add_relu(p1)
    p2 = _sc_spmm(h, src3, dst3, cpt)
    return _tc_out_proj(p2, W2)


# speedup vs baseline: 24.0568x; 1.6059x over previous
import functools

import jax
import jax.numpy as jnp
from jax import lax
from jax.experimental import pallas as pl
from jax.experimental.pallas import tpu as pltpu
from jax.experimental.pallas import tpu_sc as plsc

N_NODES = 10000
HID = 16
N_CLS = 40

NCORE = 2
NSUB = 16
NW = NCORE * NSUB
CHUNK = 125
NPAD = 10112
RPT = NPAD // NSUB


def _sc_spmm(feat, src3, dst3, cpt):
    mesh = plsc.VectorSubcoreMesh(core_axis_name="c", subcore_axis_name="s")

    assert cpt % 2 == 0, "double-buffered loop assumes an even chunk count"
    fpt = N_NODES // NSUB

    @functools.partial(
        pl.kernel,
        out_type=jax.ShapeDtypeStruct((NCORE, NPAD, HID), jnp.float32),
        mesh=mesh,
        scratch_types=[
            pltpu.VMEM((cpt, CHUNK), jnp.int32),
            pltpu.VMEM((cpt, CHUNK), jnp.int32),
            pltpu.VMEM((CHUNK, HID), jnp.float32),
            pltpu.VMEM((CHUNK, HID), jnp.float32),
            pltpu.VMEM((RPT, HID), jnp.float32),
            pltpu.VMEM_SHARED((NPAD, HID), jnp.float32),
            pltpu.VMEM_SHARED((N_NODES, HID), jnp.float32),
            pltpu.SemaphoreType.DMA,
            pltpu.SemaphoreType.DMA,
            pltpu.SemaphoreType.DMA,
            pltpu.SemaphoreType.DMA,
        ],
        compiler_params=pltpu.CompilerParams(use_tc_tiling_on_sc=False),
    )
    def spmm(feat_hbm, src_hbm, dst_hbm, out_hbm, src_v, dst_v,
             rows0_v, rows1_v, buf_v, acc_sh, feat_sh, g0, g1, s0, s1):
        c = lax.axis_index("c")
        s = lax.axis_index("s")
        w = c * NSUB + s

        @pl.loop(0, RPT)
        def _(i):
            buf_v[i, :] = jnp.zeros((HID,), jnp.float32)

        pltpu.sync_copy(buf_v, acc_sh.at[pl.ds(s * RPT, RPT)])
        pltpu.sync_copy(feat_hbm.at[pl.ds(s * fpt, fpt)],
                        feat_sh.at[pl.ds(s * fpt, fpt)])
        pltpu.sync_copy(src_hbm.at[w], src_v)
        pltpu.sync_copy(dst_hbm.at[w], dst_v)
        plsc.subcore_barrier()

        rows = (rows0_v, rows1_v)
        gsem = (g0, g1)
        ssem = (s0, s1)

        pltpu.async_copy(feat_sh.at[src_v.at[0]], rows[0], gsem[0])

        @pl.loop(0, cpt, step=2)
        def _(j):
            for b in (0, 1):
                jj = j + b
                o = 1 - b
                pltpu.make_async_copy(
                    feat_sh.at[src_v.at[jj]], rows[b], gsem[b]).wait()

                @pl.when(jj + 1 < cpt)
                def _():
                    @pl.when(jj >= 1)
                    def _():
                        pltpu.make_async_copy(
                            rows[o], acc_sh.at[dst_v.at[jj - 1]],
                            ssem[o]).wait()
                    pltpu.async_copy(
                        feat_sh.at[src_v.at[jj + 1]], rows[o], gsem[o])

                pltpu.async_copy(
                    rows[b], acc_sh.at[dst_v.at[jj]], ssem[b], add=True)

        pltpu.make_async_copy(
            rows[0], acc_sh.at[dst_v.at[cpt - 2]], ssem[0]).wait()
        pltpu.make_async_copy(
            rows[1], acc_sh.at[dst_v.at[cpt - 1]], ssem[1]).wait()
        plsc.subcore_barrier()
        pltpu.sync_copy(acc_sh.at[pl.ds(s * RPT, RPT)], buf_v)
        pltpu.sync_copy(buf_v, out_hbm.at[c, pl.ds(s * RPT, RPT)])

    return spmm(feat, src3, dst3)


def _tc_in_proj(x, W1):
    def body(x_ref, w_ref, o_ref):
        o_ref[...] = jnp.dot(x_ref[...], w_ref[...],
                             preferred_element_type=jnp.float32)

    return pl.pallas_call(
        body,
        out_shape=jax.ShapeDtypeStruct((N_NODES, HID), jnp.float32),
    )(x, W1)


def _tc_add_relu(p):
    def body(p_ref, o_ref):
        o_ref[...] = jnp.maximum(p_ref[0, :N_NODES, :] + p_ref[1, :N_NODES, :],
                                 0.0)

    return pl.pallas_call(
        body,
        out_shape=jax.ShapeDtypeStruct((N_NODES, HID), jnp.float32),
    )(p)


def _tc_out_proj(q, W2):
    def body(q_ref, w_ref, o_ref):
        z = q_ref[0, :N_NODES, :] + q_ref[1, :N_NODES, :]
        logits = jnp.dot(z, w_ref[...], preferred_element_type=jnp.float32)
        m = jnp.max(logits, axis=1, keepdims=True)
        e = jnp.exp(logits - m)
        lse = jnp.log(jnp.sum(e, axis=1, keepdims=True)) + m
        o_ref[...] = logits - lse

    return pl.pallas_call(
        body,
        out_shape=jax.ShapeDtypeStruct((N_NODES, N_CLS), jnp.float32),
    )(q, W2)


def kernel(x, edge_index, W1, W2):
    n_edges = edge_index.shape[1]
    per_worker = (n_edges + NW - 1) // NW
    cpt = (per_worker + CHUNK - 1) // CHUNK
    cpt += cpt % 2
    epad = NW * cpt * CHUNK

    src = edge_index[0].astype(jnp.int32)
    dst = edge_index[1].astype(jnp.int32)
    pad = epad - n_edges
    if pad:
        src = jnp.concatenate([src, jnp.zeros((pad,), jnp.int32)])
        dst = jnp.concatenate([dst, jnp.full((pad,), N_NODES, jnp.int32)])
    src3 = src.reshape(NW, cpt, CHUNK)
    dst3 = dst.reshape(NW, cpt, CHUNK)

    t1 = _tc_in_proj(x, W1)
    p1 = _sc_spmm(t1, src3, dst3, cpt)
    h = _tc_add_relu(p1)
    p2 = _sc_spmm(h, src3, dst3, cpt)
    return _tc_out_proj(p2, W2)


# speedup vs baseline: 28.8300x; 1.1984x over previous
import functools

import jax
import jax.numpy as jnp
from jax import lax
from jax.experimental import pallas as pl
from jax.experimental.pallas import tpu as pltpu
from jax.experimental.pallas import tpu_sc as plsc

N_NODES = 10000
HID = 16
N_CLS = 40

NCORE = 2
NSUB = 16
NW = NCORE * NSUB
CHUNK = 125
NPAD = 10112
RPT = NPAD // NSUB
FPT = N_NODES // NSUB


def _sc_spmm(feat, idx4, cpt, fuse_add_relu):
    mesh = plsc.VectorSubcoreMesh(core_axis_name="c", subcore_axis_name="s")
    assert cpt % 2 == 0, "double-buffered loop assumes an even chunk count"

    @functools.partial(
        pl.kernel,
        out_type=jax.ShapeDtypeStruct((NCORE, NPAD, HID), jnp.float32),
        mesh=mesh,
        scratch_types=[
            pltpu.VMEM((cpt, CHUNK), jnp.int32),
            pltpu.VMEM((cpt, CHUNK), jnp.int32),
            pltpu.VMEM((CHUNK, HID), jnp.float32),
            pltpu.VMEM((CHUNK, HID), jnp.float32),
            pltpu.VMEM((RPT, HID), jnp.float32),
            pltpu.VMEM((RPT, HID), jnp.float32),
            pltpu.VMEM_SHARED((NPAD, HID), jnp.float32),
            pltpu.VMEM_SHARED((NPAD, HID), jnp.float32),
            pltpu.SemaphoreType.DMA,
            pltpu.SemaphoreType.DMA,
            pltpu.SemaphoreType.DMA,
            pltpu.SemaphoreType.DMA,
        ],
        compiler_params=pltpu.CompilerParams(use_tc_tiling_on_sc=False),
    )
    def spmm(feat_hbm, idx_hbm, out_hbm, src_v, dst_v,
             rows0_v, rows1_v, buf_v, buf2_v, acc_sh, feat_sh,
             g0, g1, s0, s1):
        c = lax.axis_index("c")
        s = lax.axis_index("s")
        w = c * NSUB + s

        @pl.loop(0, RPT)
        def _(i):
            buf_v[i, :] = jnp.zeros((HID,), jnp.float32)

        pltpu.sync_copy(buf_v, acc_sh.at[pl.ds(s * RPT, RPT)])

        if fuse_add_relu:
            pltpu.sync_copy(feat_hbm.at[0, pl.ds(s * RPT, RPT)], buf_v)
            pltpu.sync_copy(feat_hbm.at[1, pl.ds(s * RPT, RPT)], buf2_v)

            @pl.loop(0, RPT)
            def _(i):
                buf_v[i, :] = jnp.maximum(buf_v[i, :] + buf2_v[i, :], 0.0)

            pltpu.sync_copy(buf_v, feat_sh.at[pl.ds(s * RPT, RPT)])
        else:
            pltpu.sync_copy(feat_hbm.at[pl.ds(s * FPT, FPT)],
                            feat_sh.at[pl.ds(s * FPT, FPT)])

        pltpu.sync_copy(idx_hbm.at[0, w], src_v)
        pltpu.sync_copy(idx_hbm.at[1, w], dst_v)
        plsc.subcore_barrier()

        rows = (rows0_v, rows1_v)
        gsem = (g0, g1)
        ssem = (s0, s1)

        pltpu.async_copy(feat_sh.at[src_v.at[0]], rows[0], gsem[0])

        @pl.loop(0, cpt, step=2)
        def _(j):
            for b in (0, 1):
                jj = j + b
                o = 1 - b
                pltpu.make_async_copy(
                    feat_sh.at[src_v.at[jj]], rows[b], gsem[b]).wait()

                @pl.when(jj + 1 < cpt)
                def _():
                    @pl.when(jj >= 1)
                    def _():
                        pltpu.make_async_copy(
                            rows[o], acc_sh.at[dst_v.at[jj - 1]],
                            ssem[o]).wait()
                    pltpu.async_copy(
                        feat_sh.at[src_v.at[jj + 1]], rows[o], gsem[o])

                pltpu.async_copy(
                    rows[b], acc_sh.at[dst_v.at[jj]], ssem[b], add=True)

        pltpu.make_async_copy(
            rows[0], acc_sh.at[dst_v.at[cpt - 2]], ssem[0]).wait()
        pltpu.make_async_copy(
            rows[1], acc_sh.at[dst_v.at[cpt - 1]], ssem[1]).wait()
        plsc.subcore_barrier()
        pltpu.sync_copy(acc_sh.at[pl.ds(s * RPT, RPT)], buf_v)
        pltpu.sync_copy(buf_v, out_hbm.at[c, pl.ds(s * RPT, RPT)])

    return spmm(feat, idx4)


def _tc_in_proj(x, W1):
    def body(x_ref, w_ref, o_ref):
        o_ref[...] = jnp.dot(x_ref[...], w_ref[...],
                             preferred_element_type=jnp.float32)

    return pl.pallas_call(
        body,
        out_shape=jax.ShapeDtypeStruct((N_NODES, HID), jnp.float32),
    )(x, W1)


def _tc_out_proj(q, W2):
    def body(q_ref, w_ref, o_ref):
        z = q_ref[0, :N_NODES, :] + q_ref[1, :N_NODES, :]
        logits = jnp.dot(z, w_ref[...], preferred_element_type=jnp.float32)
        m = jnp.max(logits, axis=1, keepdims=True)
        e = jnp.exp(logits - m)
        lse = jnp.log(jnp.sum(e, axis=1, keepdims=True)) + m
        o_ref[...] = logits - lse

    return pl.pallas_call(
        body,
        out_shape=jax.ShapeDtypeStruct((N_NODES, N_CLS), jnp.float32),
    )(q, W2)


def kernel(x, edge_index, W1, W2):
    n_edges = edge_index.shape[1]
    per_worker = (n_edges + NW - 1) // NW
    cpt = (per_worker + CHUNK - 1) // CHUNK
    cpt += cpt % 2
    epad = NW * cpt * CHUNK

    idx = edge_index.astype(jnp.int32)
    pad = epad - n_edges
    if pad:
        fill = jnp.stack([jnp.zeros((pad,), jnp.int32),
                          jnp.full((pad,), N_NODES, jnp.int32)])
        idx = jnp.concatenate([idx, fill], axis=1)
    idx4 = idx.reshape(2, NW, cpt, CHUNK)

    t1 = _sc_spmm(_tc_in_proj(x, W1), idx4, cpt, False)
    p2 = _sc_spmm(t1, idx4, cpt, True)
    return _tc_out_proj(p2, W2)


# speedup vs baseline: 30.8503x; 1.0701x over previous
import functools

import jax
import jax.numpy as jnp
from jax import lax
from jax.experimental import pallas as pl
from jax.experimental.pallas import tpu as pltpu
from jax.experimental.pallas import tpu_sc as plsc

N_NODES = 10000
HID = 16
N_CLS = 40

NCORE = 2
NSUB = 16
NW = NCORE * NSUB
CHUNK = 125
NPAD = 10112
RPT = NPAD // NSUB
FPT = N_NODES // NSUB


def _sc_spmm(feat, idx4, cpt, fuse_add_relu):
    mesh = plsc.VectorSubcoreMesh(core_axis_name="c", subcore_axis_name="s")
    NBUF = 4
    assert cpt % NBUF == 0, "pipeline assumes chunk count divisible by NBUF"

    @functools.partial(
        pl.kernel,
        out_type=jax.ShapeDtypeStruct((NCORE, NPAD, HID), jnp.float32),
        mesh=mesh,
        scratch_types=[
            pltpu.VMEM((cpt, CHUNK), jnp.int32),
            pltpu.VMEM((cpt, CHUNK), jnp.int32),
            [pltpu.VMEM((CHUNK, HID), jnp.float32)] * NBUF,
            pltpu.VMEM((RPT, HID), jnp.float32),
            pltpu.VMEM((RPT, HID), jnp.float32),
            pltpu.VMEM((RPT, HID), jnp.float32),
            pltpu.VMEM_SHARED((NPAD, HID), jnp.float32),
            pltpu.VMEM_SHARED((NPAD, HID), jnp.float32),
            [pltpu.SemaphoreType.DMA] * NBUF,
            [pltpu.SemaphoreType.DMA] * NBUF,
            pltpu.SemaphoreType.DMA,
        ],
        compiler_params=pltpu.CompilerParams(use_tc_tiling_on_sc=False),
    )
    def spmm(feat_hbm, idx_hbm, out_hbm, src_v, dst_v,
             rows, buf_v, buf2_v, buf3_v, acc_sh, feat_sh,
             gsem, ssem, stage_sem):
        c = lax.axis_index("c")
        s = lax.axis_index("s")
        w = c * NSUB + s

        pltpu.async_copy(idx_hbm.at[0, w], src_v, gsem[0])
        pltpu.async_copy(idx_hbm.at[1, w], dst_v, gsem[1])

        @pl.loop(0, RPT)
        def _(i):
            buf_v[i, :] = jnp.zeros((HID,), jnp.float32)

        pltpu.async_copy(buf_v, acc_sh.at[pl.ds(s * RPT, RPT)], ssem[0])

        if fuse_add_relu:
            pltpu.async_copy(feat_hbm.at[0, pl.ds(s * RPT, RPT)], buf2_v,
                             gsem[2])
            pltpu.async_copy(feat_hbm.at[1, pl.ds(s * RPT, RPT)], buf3_v,
                             gsem[3])
            pltpu.make_async_copy(feat_hbm.at[0, pl.ds(s * RPT, RPT)], buf2_v,
                                  gsem[2]).wait()
            pltpu.make_async_copy(feat_hbm.at[1, pl.ds(s * RPT, RPT)], buf3_v,
                                  gsem[3]).wait()

            @pl.loop(0, RPT)
            def _(i):
                buf2_v[i, :] = jnp.maximum(buf2_v[i, :] + buf3_v[i, :], 0.0)

            pltpu.sync_copy(buf2_v, feat_sh.at[pl.ds(s * RPT, RPT)])
        else:
            pltpu.async_copy(feat_hbm.at[pl.ds(s * FPT, FPT)],
                             feat_sh.at[pl.ds(s * FPT, FPT)], stage_sem)
            pltpu.make_async_copy(feat_hbm.at[pl.ds(s * FPT, FPT)],
                                  feat_sh.at[pl.ds(s * FPT, FPT)],
                                  stage_sem).wait()

        pltpu.make_async_copy(idx_hbm.at[0, w], src_v, gsem[0]).wait()
        pltpu.make_async_copy(idx_hbm.at[1, w], dst_v, gsem[1]).wait()
        pltpu.make_async_copy(buf_v, acc_sh.at[pl.ds(s * RPT, RPT)],
                              ssem[0]).wait()
        plsc.subcore_barrier()

        for b in range(NBUF - 1):
            pltpu.async_copy(feat_sh.at[src_v.at[b]], rows[b], gsem[b])

        @pl.loop(0, cpt, step=NBUF)
        def _(j):
            for b in range(NBUF):
                jj = j + b
                nxt = jj + NBUF - 1
                pb = (b + NBUF - 1) % NBUF
                pltpu.make_async_copy(
                    feat_sh.at[src_v.at[jj]], rows[b], gsem[b]).wait()

                @pl.when(nxt < cpt)
                def _():
                    @pl.when(jj >= 1)
                    def _():
                        pltpu.make_async_copy(
                            rows[pb], acc_sh.at[dst_v.at[jj - 1]],
                            ssem[pb]).wait()
                    pltpu.async_copy(
                        feat_sh.at[src_v.at[nxt]], rows[pb], gsem[pb])

                pltpu.async_copy(
                    rows[b], acc_sh.at[dst_v.at[jj]], ssem[b], add=True)

        for b in range(NBUF):
            pltpu.make_async_copy(
                rows[b], acc_sh.at[dst_v.at[cpt - NBUF + b]], ssem[b]).wait()
        plsc.subcore_barrier()
        pltpu.sync_copy(acc_sh.at[pl.ds(s * RPT, RPT)], buf_v)
        pltpu.sync_copy(buf_v, out_hbm.at[c, pl.ds(s * RPT, RPT)])

    return spmm(feat, idx4)


def _tc_in_proj(x, W1):
    def body(x_ref, w_ref, o_ref):
        o_ref[...] = jnp.dot(x_ref[...], w_ref[...],
                             preferred_element_type=jnp.float32)

    return pl.pallas_call(
        body,
        out_shape=jax.ShapeDtypeStruct((N_NODES, HID), jnp.float32),
    )(x, W1)


def _tc_out_proj(q, W2):
    def body(q_ref, w_ref, o_ref):
        z = q_ref[0, :N_NODES, :] + q_ref[1, :N_NODES, :]
        logits = jnp.dot(z, w_ref[...], preferred_element_type=jnp.float32)
        m = jnp.max(logits, axis=1, keepdims=True)
        e = jnp.exp(logits - m)
        lse = jnp.log(jnp.sum(e, axis=1, keepdims=True)) + m
        o_ref[...] = logits - lse

    return pl.pallas_call(
        body,
        out_shape=jax.ShapeDtypeStruct((N_NODES, N_CLS), jnp.float32),
    )(q, W2)


def kernel(x, edge_index, W1, W2):
    n_edges = edge_index.shape[1]
    per_worker = (n_edges + NW - 1) // NW
    cpt = (per_worker + CHUNK - 1) // CHUNK
    cpt += cpt % 2
    epad = NW * cpt * CHUNK

    idx = edge_index.astype(jnp.int32)
    pad = epad - n_edges
    if pad:
        fill = jnp.stack([jnp.zeros((pad,), jnp.int32),
                          jnp.full((pad,), N_NODES, jnp.int32)])
        idx = jnp.concatenate([idx, fill], axis=1)
    idx4 = idx.reshape(2, NW, cpt, CHUNK)

    t1 = _sc_spmm(_tc_in_proj(x, W1), idx4, cpt, False)
    p2 = _sc_spmm(t1, idx4, cpt, True)
    return _tc_out_proj(p2, W2)


# speedup vs baseline: 30.9325x; 1.0027x over previous
import functools

import jax
import jax.numpy as jnp
from jax import lax
from jax.experimental import pallas as pl
from jax.experimental.pallas import tpu as pltpu
from jax.experimental.pallas import tpu_sc as plsc

N_NODES = 10000
HID = 16
N_CLS = 40

NCORE = 2
NSUB = 16
NW = NCORE * NSUB
CHUNK = 125
NPAD = 10112
RPT = NPAD // NSUB
FPT = N_NODES // NSUB


def _sc_spmm(feat, idx4, cpt, fuse_add_relu):
    mesh = plsc.VectorSubcoreMesh(core_axis_name="c", subcore_axis_name="s")
    NBUF = 8
    assert cpt % NBUF == 0, "pipeline assumes chunk count divisible by NBUF"

    @functools.partial(
        pl.kernel,
        out_type=jax.ShapeDtypeStruct((NCORE, NPAD, HID), jnp.float32),
        mesh=mesh,
        scratch_types=[
            pltpu.VMEM((cpt, CHUNK), jnp.int32),
            pltpu.VMEM((cpt, CHUNK), jnp.int32),
            [pltpu.VMEM((CHUNK, HID), jnp.float32)] * NBUF,
            pltpu.VMEM((RPT, HID), jnp.float32),
            pltpu.VMEM((RPT, HID), jnp.float32),
            pltpu.VMEM((RPT, HID), jnp.float32),
            pltpu.VMEM_SHARED((NPAD, HID), jnp.float32),
            pltpu.VMEM_SHARED((NPAD, HID), jnp.float32),
            [pltpu.SemaphoreType.DMA] * NBUF,
            [pltpu.SemaphoreType.DMA] * NBUF,
            pltpu.SemaphoreType.DMA,
        ],
        compiler_params=pltpu.CompilerParams(use_tc_tiling_on_sc=False),
    )
    def spmm(feat_hbm, idx_hbm, out_hbm, src_v, dst_v,
             rows, buf_v, buf2_v, buf3_v, acc_sh, feat_sh,
             gsem, ssem, stage_sem):
        c = lax.axis_index("c")
        s = lax.axis_index("s")
        w = c * NSUB + s

        pltpu.async_copy(idx_hbm.at[0, w], src_v, gsem[0])
        pltpu.async_copy(idx_hbm.at[1, w], dst_v, gsem[1])

        @pl.loop(0, RPT)
        def _(i):
            buf_v[i, :] = jnp.zeros((HID,), jnp.float32)

        pltpu.async_copy(buf_v, acc_sh.at[pl.ds(s * RPT, RPT)], ssem[0])

        if fuse_add_relu:
            pltpu.async_copy(feat_hbm.at[0, pl.ds(s * RPT, RPT)], buf2_v,
                             gsem[2])
            pltpu.async_copy(feat_hbm.at[1, pl.ds(s * RPT, RPT)], buf3_v,
                             gsem[3])
            pltpu.make_async_copy(feat_hbm.at[0, pl.ds(s * RPT, RPT)], buf2_v,
                                  gsem[2]).wait()
            pltpu.make_async_copy(feat_hbm.at[1, pl.ds(s * RPT, RPT)], buf3_v,
                                  gsem[3]).wait()

            @pl.loop(0, RPT)
            def _(i):
                buf2_v[i, :] = jnp.maximum(buf2_v[i, :] + buf3_v[i, :], 0.0)

            pltpu.sync_copy(buf2_v, feat_sh.at[pl.ds(s * RPT, RPT)])
        else:
            pltpu.async_copy(feat_hbm.at[pl.ds(s * FPT, FPT)],
                             feat_sh.at[pl.ds(s * FPT, FPT)], stage_sem)
            pltpu.make_async_copy(feat_hbm.at[pl.ds(s * FPT, FPT)],
                                  feat_sh.at[pl.ds(s * FPT, FPT)],
                                  stage_sem).wait()

        pltpu.make_async_copy(idx_hbm.at[0, w], src_v, gsem[0]).wait()
        pltpu.make_async_copy(idx_hbm.at[1, w], dst_v, gsem[1]).wait()
        pltpu.make_async_copy(buf_v, acc_sh.at[pl.ds(s * RPT, RPT)],
                              ssem[0]).wait()
        plsc.subcore_barrier()

        for b in range(NBUF - 1):
            pltpu.async_copy(feat_sh.at[src_v.at[b]], rows[b], gsem[b])

        @pl.loop(0, cpt, step=NBUF)
        def _(j):
            for b in range(NBUF):
                jj = j + b
                nxt = jj + NBUF - 1
                pb = (b + NBUF - 1) % NBUF
                pltpu.make_async_copy(
                    feat_sh.at[src_v.at[jj]], rows[b], gsem[b]).wait()

                @pl.when(nxt < cpt)
                def _():
                    @pl.when(jj >= 1)
                    def _():
                        pltpu.make_async_copy(
                            rows[pb], acc_sh.at[dst_v.at[jj - 1]],
                            ssem[pb]).wait()
                    pltpu.async_copy(
                        feat_sh.at[src_v.at[nxt]], rows[pb], gsem[pb])

                pltpu.async_copy(
                    rows[b], acc_sh.at[dst_v.at[jj]], ssem[b], add=True)

        for b in range(NBUF):
            pltpu.make_async_copy(
                rows[b], acc_sh.at[dst_v.at[cpt - NBUF + b]], ssem[b]).wait()
        plsc.subcore_barrier()
        pltpu.sync_copy(acc_sh.at[pl.ds(s * RPT, RPT)], buf_v)
        pltpu.sync_copy(buf_v, out_hbm.at[c, pl.ds(s * RPT, RPT)])

    return spmm(feat, idx4)


def _tc_in_proj(x, W1):
    def body(x_ref, w_ref, o_ref):
        o_ref[...] = jnp.dot(x_ref[...], w_ref[...],
                             preferred_element_type=jnp.float32)

    return pl.pallas_call(
        body,
        out_shape=jax.ShapeDtypeStruct((N_NODES, HID), jnp.float32),
    )(x, W1)


def _tc_out_proj(q, W2):
    def body(q_ref, w_ref, o_ref):
        z = q_ref[0, :N_NODES, :] + q_ref[1, :N_NODES, :]
        logits = jnp.dot(z, w_ref[...], preferred_element_type=jnp.float32)
        m = jnp.max(logits, axis=1, keepdims=True)
        e = jnp.exp(logits - m)
        lse = jnp.log(jnp.sum(e, axis=1, keepdims=True)) + m
        o_ref[...] = logits - lse

    return pl.pallas_call(
        body,
        out_shape=jax.ShapeDtypeStruct((N_NODES, N_CLS), jnp.float32),
    )(q, W2)


def kernel(x, edge_index, W1, W2):
    n_edges = edge_index.shape[1]
    per_worker = (n_edges + NW - 1) // NW
    cpt = (per_worker + CHUNK - 1) // CHUNK
    cpt = (cpt + 7) // 8 * 8
    epad = NW * cpt * CHUNK

    idx = edge_index.astype(jnp.int32)
    pad = epad - n_edges
    if pad:
        fill = jnp.stack([jnp.zeros((pad,), jnp.int32),
                          jnp.full((pad,), N_NODES, jnp.int32)])
        idx = jnp.concatenate([idx, fill], axis=1)
    idx4 = idx.reshape(2, NW, cpt, CHUNK)

    t1 = _sc_spmm(_tc_in_proj(x, W1), idx4, cpt, False)
    p2 = _sc_spmm(t1, idx4, cpt, True)
    return _tc_out_proj(p2, W2)
